# Initial kernel scaffold; baseline (speedup 1.0000x reference)
import jax
import jax.numpy as jnp
from jax.experimental import pallas as pl


def kernel(top_vecs, position_ids, pos_table):
    raise NotImplementedError("write your pallas kernel here")


# speedup vs baseline: 4.3412x; 4.3412x over previous
import functools

import jax
import jax.numpy as jnp
from jax import lax
from jax.experimental import pallas as pl
from jax.experimental.pallas import tpu as pltpu
from jax.experimental.pallas import tpu_sc as plsc

HIDDEN = 128
CHUNK = 400


@functools.lru_cache(maxsize=None)
def _build_gather(total, hidden):
    info = plsc.get_sparse_core_info()
    nw = info.num_cores * info.num_subcores
    per_w = total // nw
    n_chunks = per_w // CHUNK
    mesh = plsc.VectorSubcoreMesh(core_axis_name="c", subcore_axis_name="s")

    @functools.partial(
        pl.kernel,
        mesh=mesh,
        out_type=jax.ShapeDtypeStruct((total, hidden), jnp.float32),
        scratch_types=[
            pltpu.VMEM((per_w,), jnp.int32),
            pltpu.VMEM((CHUNK, hidden), jnp.float32),
            pltpu.SemaphoreType.DMA,
        ],
    )
    def gather_kernel(table_hbm, idx_hbm, out_hbm, idx_v, rows_v, sem):
        wid = lax.axis_index("s") * info.num_cores + lax.axis_index("c")
        base = wid * per_w
        pltpu.sync_copy(idx_hbm.at[pl.ds(base, per_w)], idx_v)

        def body(i, carry):
            off = i * CHUNK
            pltpu.async_copy(
                table_hbm.at[idx_v.at[pl.ds(off, CHUNK)]], rows_v, sem
            ).wait()
            pltpu.sync_copy(rows_v, out_hbm.at[pl.ds(base + off, CHUNK)])
            return carry

        lax.fori_loop(0, n_chunks, body, 0)

    return gather_kernel


def kernel(top_vecs, position_ids, pos_table):
    del top_vecs
    b, s = position_ids.shape
    idx = position_ids.reshape(-1).astype(jnp.int32)
    out = _build_gather(b * s, pos_table.shape[1])(pos_table, idx)
    return out.reshape(b, s, pos_table.shape[1])


# speedup vs baseline: 4.3880x; 1.0108x over previous
import functools

import jax
import jax.numpy as jnp
from jax import lax
from jax.experimental import pallas as pl
from jax.experimental.pallas import tpu as pltpu
from jax.experimental.pallas import tpu_sc as plsc

CHUNK = 400


@functools.lru_cache(maxsize=None)
def _build_gather(total, hidden):
    info = plsc.get_sparse_core_info()
    nw = info.num_cores * info.num_subcores
    per_w = total // nw
    n_chunks = per_w // CHUNK
    assert n_chunks % 2 == 0
    mesh = plsc.VectorSubcoreMesh(core_axis_name="c", subcore_axis_name="s")

    @functools.partial(
        pl.kernel,
        mesh=mesh,
        out_type=jax.ShapeDtypeStruct((total, hidden), jnp.float32),
        scratch_types=[
            pltpu.VMEM((per_w,), jnp.int32),
            pltpu.VMEM((CHUNK, hidden), jnp.float32),
            pltpu.VMEM((CHUNK, hidden), jnp.float32),
            pltpu.SemaphoreType.DMA,
            pltpu.SemaphoreType.DMA,
        ],
    )
    def gather_kernel(table_hbm, idx_hbm, out_hbm, idx_v, rows0, rows1,
                      sem_g, sem_o):
        wid = lax.axis_index("s") * info.num_cores + lax.axis_index("c")
        base = wid * per_w
        pltpu.sync_copy(idx_hbm.at[pl.ds(base, per_w)], idx_v)
        bufs = (rows0, rows1)

        def start_gather(i, buf):
            pltpu.async_copy(table_hbm.at[idx_v.at[pl.ds(i * CHUNK, CHUNK)]],
                             buf, sem_g)

        def wait_gather(i, buf):
            pltpu.make_async_copy(
                table_hbm.at[idx_v.at[pl.ds(i * CHUNK, CHUNK)]], buf, sem_g
            ).wait()

        def start_out(i, buf):
            pltpu.async_copy(buf, out_hbm.at[pl.ds(base + i * CHUNK, CHUNK)],
                             sem_o)

        def wait_out(i, buf):
            pltpu.make_async_copy(
                buf, out_hbm.at[pl.ds(base + i * CHUNK, CHUNK)], sem_o
            ).wait()

        start_gather(0, bufs[0])

        def pair_body(p, carry):
            for b in range(2):
                i = 2 * p + b
                buf, alt = bufs[b], bufs[1 - b]
                wait_gather(i, buf)
                if b == 0:
                    @pl.when(p > 0)
                    def _():
                        wait_out(i - 1, alt)
                else:
                    wait_out(i - 1, alt)
                is_last = jnp.logical_and(p == n_chunks // 2 - 1, b == 1)

                @pl.when(jnp.logical_not(is_last))
                def _():
                    start_gather(i + 1, alt)

                start_out(i, buf)
            return carry

        lax.fori_loop(0, n_chunks // 2, pair_body, 0)
        wait_out(n_chunks - 1, bufs[1])

    return gather_kernel


def kernel(top_vecs, position_ids, pos_table):
    del top_vecs
    b, s = position_ids.shape
    idx = position_ids.reshape(-1).astype(jnp.int32)
    out = _build_gather(b * s, pos_table.shape[1])(pos_table, idx)
    return out.reshape(b, s, pos_table.shape[1])


# speedup vs baseline: 11.6038x; 2.6445x over previous
import functools

import jax
import jax.numpy as jnp
from jax import lax
from jax.experimental import pallas as pl
from jax.experimental.pallas import tpu as pltpu
from jax.experimental.pallas import tpu_sc as plsc

CHUNK = 400


@functools.lru_cache(maxsize=None)
def _build_gather(total, n_rows, hidden):
    info = plsc.get_sparse_core_info()
    nc, ns = info.num_cores, info.num_subcores
    nw = nc * ns
    per_w = total // nw
    n_chunks = per_w // CHUNK
    assert n_chunks % 2 == 0
    rows_per_tile = n_rows // ns
    mesh = plsc.VectorSubcoreMesh(core_axis_name="c", subcore_axis_name="s")

    @functools.partial(
        pl.kernel,
        mesh=mesh,
        out_type=jax.ShapeDtypeStruct((total, hidden), jnp.float32),
        scratch_types=[
            pltpu.VMEM((per_w,), jnp.int32),
            pltpu.VMEM((CHUNK, hidden), jnp.float32),
            pltpu.VMEM((CHUNK, hidden), jnp.float32),
            pltpu.VMEM_SHARED((n_rows, hidden), jnp.float32),
            pltpu.SemaphoreType.DMA,
            pltpu.SemaphoreType.DMA,
        ],
    )
    def gather_kernel(table_hbm, idx_hbm, out_hbm, idx_v, rows0, rows1,
                      table_sp, sem_g, sem_o):
        cid = lax.axis_index("c")
        sid = lax.axis_index("s")
        wid = sid * nc + cid
        base = wid * per_w

        stripe = sid * rows_per_tile
        bounce = rows0.at[pl.ds(0, rows_per_tile)]
        pltpu.sync_copy(table_hbm.at[pl.ds(stripe, rows_per_tile)], bounce)
        pltpu.sync_copy(bounce, table_sp.at[pl.ds(stripe, rows_per_tile)])
        pltpu.sync_copy(idx_hbm.at[pl.ds(base, per_w)], idx_v)
        plsc.subcore_barrier()

        bufs = (rows0, rows1)

        def start_gather(i, buf):
            pltpu.async_copy(table_sp.at[idx_v.at[pl.ds(i * CHUNK, CHUNK)]],
                             buf, sem_g)

        def wait_gather(i, buf):
            pltpu.make_async_copy(
                table_sp.at[idx_v.at[pl.ds(i * CHUNK, CHUNK)]], buf, sem_g
            ).wait()

        def start_out(i, buf):
            pltpu.async_copy(buf, out_hbm.at[pl.ds(base + i * CHUNK, CHUNK)],
                             sem_o)

        def wait_out(i, buf):
            pltpu.make_async_copy(
                buf, out_hbm.at[pl.ds(base + i * CHUNK, CHUNK)], sem_o
            ).wait()

        start_gather(0, bufs[0])

        def pair_body(p, carry):
            for b in range(2):
                i = 2 * p + b
                buf, alt = bufs[b], bufs[1 - b]
                wait_gather(i, buf)
                if b == 0:
                    @pl.when(p > 0)
                    def _():
                        wait_out(i - 1, alt)
                else:
                    wait_out(i - 1, alt)
                is_last = jnp.logical_and(p == n_chunks // 2 - 1, b == 1)

                @pl.when(jnp.logical_not(is_last))
                def _():
                    start_gather(i + 1, alt)

                start_out(i, buf)
            return carry

        lax.fori_loop(0, n_chunks // 2, pair_body, 0)
        wait_out(n_chunks - 1, bufs[1])

    return gather_kernel


def kernel(top_vecs, position_ids, pos_table):
    del top_vecs
    b, s = position_ids.shape
    idx = position_ids.reshape(-1).astype(jnp.int32)
    out = _build_gather(b * s, pos_table.shape[0], pos_table.shape[1])(
        pos_table, idx)
    return out.reshape(b, s, pos_table.shape[1])


# speedup vs baseline: 11.7468x; 1.0123x over previous
import functools

import jax
import jax.numpy as jnp
from jax import lax
from jax.experimental import pallas as pl
from jax.experimental.pallas import tpu as pltpu
from jax.experimental.pallas import tpu_sc as plsc

CHUNK = 400


@functools.lru_cache(maxsize=None)
def _build_gather(total, n_rows, hidden):
    info = plsc.get_sparse_core_info()
    nc, ns = info.num_cores, info.num_subcores
    nw = nc * ns
    per_w = total // nw
    n_chunks = per_w // CHUNK
    assert n_chunks % 2 == 0
    rows_per_tile = n_rows // ns
    mesh = plsc.VectorSubcoreMesh(core_axis_name="c", subcore_axis_name="s")

    @functools.partial(
        pl.kernel,
        mesh=mesh,
        out_type=jax.ShapeDtypeStruct((total, hidden), jnp.float32),
        scratch_types=[
            pltpu.VMEM((per_w,), jnp.int32),
            pltpu.VMEM((CHUNK, hidden), jnp.float32),
            pltpu.VMEM((CHUNK, hidden), jnp.float32),
            pltpu.VMEM_SHARED((n_rows, hidden), jnp.float32),
            pltpu.SemaphoreType.DMA,
            pltpu.SemaphoreType.DMA,
        ],
    )
    def gather_kernel(table_hbm, idx_hbm, out_hbm, idx_v, rows0, rows1,
                      table_sp, sem_g, sem_o):
        cid = lax.axis_index("c")
        sid = lax.axis_index("s")
        wid = sid * nc + cid
        base = wid * per_w

        idx_cp = pltpu.make_async_copy(idx_hbm.at[pl.ds(base, per_w)], idx_v,
                                       sem_o)
        idx_cp.start()
        stripe = sid * rows_per_tile
        bounce = rows1.at[pl.ds(0, rows_per_tile)]
        pltpu.sync_copy(table_hbm.at[pl.ds(stripe, rows_per_tile)], bounce)
        pltpu.sync_copy(bounce, table_sp.at[pl.ds(stripe, rows_per_tile)])
        idx_cp.wait()
        plsc.subcore_barrier()

        bufs = (rows0, rows1)

        def start_gather(i, buf):
            pltpu.async_copy(table_sp.at[idx_v.at[pl.ds(i * CHUNK, CHUNK)]],
                             buf, sem_g)

        def wait_gather(i, buf):
            pltpu.make_async_copy(
                table_sp.at[idx_v.at[pl.ds(i * CHUNK, CHUNK)]], buf, sem_g
            ).wait()

        def start_out(i, buf):
            pltpu.async_copy(buf, out_hbm.at[pl.ds(base + i * CHUNK, CHUNK)],
                             sem_o)

        def wait_out(i, buf):
            pltpu.make_async_copy(
                buf, out_hbm.at[pl.ds(base + i * CHUNK, CHUNK)], sem_o
            ).wait()

        start_gather(0, bufs[0])

        def pair_body(p, carry):
            for b in range(2):
                i = 2 * p + b
                buf, alt = bufs[b], bufs[1 - b]
                wait_gather(i, buf)
                if b == 0:
                    @pl.when(p > 0)
                    def _():
                        wait_out(i - 1, alt)
                else:
                    wait_out(i - 1, alt)
                is_last = jnp.logical_and(p == n_chunks // 2 - 1, b == 1)

                @pl.when(jnp.logical_not(is_last))
                def _():
                    start_gather(i + 1, alt)

                start_out(i, buf)
            return carry

        lax.fori_loop(0, n_chunks // 2, pair_body, 0)
        wait_out(n_chunks - 1, bufs[1])

    return gather_kernel


def kernel(top_vecs, position_ids, pos_table):
    del top_vecs
    b, s = position_ids.shape
    idx = position_ids.reshape(-1).astype(jnp.int32)
    out = _build_gather(b * s, pos_table.shape[0], pos_table.shape[1])(
        pos_table, idx)
    return out.reshape(b, s, pos_table.shape[1])


# speedup vs baseline: 11.7534x; 1.0006x over previous
import functools

import jax
import jax.numpy as jnp
from jax import lax
from jax.experimental import pallas as pl
from jax.experimental.pallas import tpu as pltpu
from jax.experimental.pallas import tpu_sc as plsc

CHUNK = 400


@functools.lru_cache(maxsize=None)
def _build_gather(total, n_rows, hidden):
    info = plsc.get_sparse_core_info()
    nc, ns = info.num_cores, info.num_subcores
    nw = nc * ns
    per_w = total // nw
    n_chunks = per_w // CHUNK
    assert n_chunks % 2 == 0
    rows_per_tile = n_rows // ns
    mesh = plsc.VectorSubcoreMesh(core_axis_name="c", subcore_axis_name="s")

    @functools.partial(
        pl.kernel,
        mesh=mesh,
        out_type=jax.ShapeDtypeStruct((total, hidden), jnp.float32),
        scratch_types=[
            pltpu.VMEM((per_w,), jnp.int32),
            pltpu.VMEM((CHUNK, hidden), jnp.float32),
            pltpu.VMEM((CHUNK, hidden), jnp.float32),
            pltpu.VMEM_SHARED((n_rows, hidden), jnp.float32),
            pltpu.SemaphoreType.DMA,
            pltpu.SemaphoreType.DMA,
        ],
    )
    def gather_kernel(table_hbm, idx_hbm, out_hbm, idx_v, rows0, rows1,
                      table_sp, sem0, sem1):
        cid = lax.axis_index("c")
        sid = lax.axis_index("s")
        wid = sid * nc + cid
        base = wid * per_w

        idx_cp = pltpu.make_async_copy(idx_hbm.at[pl.ds(base, per_w)], idx_v,
                                       sem0)
        idx_cp.start()
        stripe = sid * rows_per_tile
        bounce = rows1.at[pl.ds(0, rows_per_tile)]
        pltpu.sync_copy(table_hbm.at[pl.ds(stripe, rows_per_tile)], bounce)
        pltpu.sync_copy(bounce, table_sp.at[pl.ds(stripe, rows_per_tile)])
        idx_cp.wait()
        plsc.subcore_barrier()

        bufs = (rows0, rows1)
        sems = (sem0, sem1)

        def start_gather(i, buf, sem):
            pltpu.async_copy(table_sp.at[idx_v.at[pl.ds(i * CHUNK, CHUNK)]],
                             buf, sem)

        def wait_gather(i, buf, sem):
            pltpu.make_async_copy(
                table_sp.at[idx_v.at[pl.ds(i * CHUNK, CHUNK)]], buf, sem
            ).wait()

        def start_out(i, buf, sem):
            pltpu.async_copy(buf, out_hbm.at[pl.ds(base + i * CHUNK, CHUNK)],
                             sem)

        def wait_out(i, buf, sem):
            pltpu.make_async_copy(
                buf, out_hbm.at[pl.ds(base + i * CHUNK, CHUNK)], sem
            ).wait()

        start_gather(0, bufs[0], sems[0])

        def pair_body(p, carry):
            for b in range(2):
                i = 2 * p + b
                buf, alt = bufs[b], bufs[1 - b]
                sem, alt_sem = sems[b], sems[1 - b]
                wait_gather(i, buf, sem)
                start_out(i, buf, sem)
                if b == 0:
                    @pl.when(p > 0)
                    def _():
                        wait_out(i - 1, alt, alt_sem)
                        start_gather(i + 1, alt, alt_sem)

                    @pl.when(p == 0)
                    def _():
                        start_gather(i + 1, alt, alt_sem)
                else:
                    wait_out(i - 1, alt, alt_sem)

                    @pl.when(i + 1 < n_chunks)
                    def _():
                        start_gather(i + 1, alt, alt_sem)
            return carry

        lax.fori_loop(0, n_chunks // 2, pair_body, 0)
        wait_out(n_chunks - 1, bufs[1], sems[1])

    return gather_kernel


def kernel(top_vecs, position_ids, pos_table):
    del top_vecs
    b, s = position_ids.shape
    idx = position_ids.reshape(-1).astype(jnp.int32)
    out = _build_gather(b * s, pos_table.shape[0], pos_table.shape[1])(
        pos_table, idx)
    return out.reshape(b, s, pos_table.shape[1])


# speedup vs baseline: 11.7714x; 1.0015x over previous
import functools

import jax
import jax.numpy as jnp
from jax import lax
from jax.experimental import pallas as pl
from jax.experimental.pallas import tpu as pltpu
from jax.experimental.pallas import tpu_sc as plsc

CHUNK = 400


@functools.lru_cache(maxsize=None)
def _build_gather(total, n_rows, hidden):
    info = plsc.get_sparse_core_info()
    nc, ns = info.num_cores, info.num_subcores
    nw = nc * ns
    per_w = total // nw
    n_chunks = per_w // CHUNK
    assert n_chunks % 2 == 0
    rows_per_tile = n_rows // ns
    mesh = plsc.VectorSubcoreMesh(core_axis_name="c", subcore_axis_name="s")

    @functools.partial(
        pl.kernel,
        mesh=mesh,
        out_type=jax.ShapeDtypeStruct((total, hidden), jnp.float32),
        scratch_types=[
            pltpu.VMEM((per_w,), jnp.int32),
            pltpu.VMEM((CHUNK, hidden), jnp.float32),
            pltpu.VMEM((CHUNK, hidden), jnp.float32),
            pltpu.VMEM_SHARED((n_rows, hidden), jnp.float32),
            pltpu.SemaphoreType.DMA,
            pltpu.SemaphoreType.DMA,
        ],
    )
    def gather_kernel(table_hbm, idx_hbm, out_hbm, idx_v, rows0, rows1,
                      table_sp, sem0, sem1):
        cid = lax.axis_index("c")
        sid = lax.axis_index("s")
        wid = sid * nc + cid
        base = wid * per_w

        idx_cp = pltpu.make_async_copy(idx_hbm.at[pl.ds(base, per_w)], idx_v,
                                       sem0)
        idx_cp.start()
        stripe = sid * rows_per_tile
        pltpu.sync_copy(table_hbm.at[pl.ds(stripe, rows_per_tile)],
                        table_sp.at[pl.ds(stripe, rows_per_tile)])
        plsc.subcore_barrier()
        idx_cp.wait()

        bufs = (rows0, rows1)
        sems = (sem0, sem1)

        def start_gather(i, buf, sem):
            pltpu.async_copy(table_sp.at[idx_v.at[pl.ds(i * CHUNK, CHUNK)]],
                             buf, sem)

        def wait_gather(i, buf, sem):
            pltpu.make_async_copy(
                table_sp.at[idx_v.at[pl.ds(i * CHUNK, CHUNK)]], buf, sem
            ).wait()

        def start_out(i, buf, sem):
            pltpu.async_copy(buf, out_hbm.at[pl.ds(base + i * CHUNK, CHUNK)],
                             sem)

        def wait_out(i, buf, sem):
            pltpu.make_async_copy(
                buf, out_hbm.at[pl.ds(base + i * CHUNK, CHUNK)], sem
            ).wait()

        start_gather(0, bufs[0], sems[0])

        def pair_body(p, carry):
            for b in range(2):
                i = 2 * p + b
                buf, alt = bufs[b], bufs[1 - b]
                sem, alt_sem = sems[b], sems[1 - b]
                wait_gather(i, buf, sem)
                start_out(i, buf, sem)
                if b == 0:
                    @pl.when(p > 0)
                    def _():
                        wait_out(i - 1, alt, alt_sem)
                        start_gather(i + 1, alt, alt_sem)

                    @pl.when(p == 0)
                    def _():
                        start_gather(i + 1, alt, alt_sem)
                else:
                    wait_out(i - 1, alt, alt_sem)

                    @pl.when(i + 1 < n_chunks)
                    def _():
                        start_gather(i + 1, alt, alt_sem)
            return carry

        lax.fori_loop(0, n_chunks // 2, pair_body, 0)
        wait_out(n_chunks - 1, bufs[1], sems[1])

    return gather_kernel


def kernel(top_vecs, position_ids, pos_table):
    del top_vecs
    b, s = position_ids.shape
    idx = position_ids.reshape(-1).astype(jnp.int32)
    out = _build_gather(b * s, pos_table.shape[0], pos_table.shape[1])(
        pos_table, idx)
    return out.reshape(b, s, pos_table.shape[1])


# speedup vs baseline: 11.7790x; 1.0006x over previous
import functools

import jax
import jax.numpy as jnp
from jax import lax
from jax.experimental import pallas as pl
from jax.experimental.pallas import tpu as pltpu
from jax.experimental.pallas import tpu_sc as plsc

CHUNK = 200


@functools.lru_cache(maxsize=None)
def _build_gather(total, n_rows, hidden):
    info = plsc.get_sparse_core_info()
    nc, ns = info.num_cores, info.num_subcores
    nw = nc * ns
    per_w = total // nw
    n_chunks = per_w // CHUNK
    assert n_chunks % 2 == 0
    rows_per_tile = n_rows // ns
    mesh = plsc.VectorSubcoreMesh(core_axis_name="c", subcore_axis_name="s")

    @functools.partial(
        pl.kernel,
        mesh=mesh,
        out_type=jax.ShapeDtypeStruct((total, hidden), jnp.float32),
        scratch_types=[
            pltpu.VMEM((per_w,), jnp.int32),
            pltpu.VMEM((CHUNK, hidden), jnp.float32),
            pltpu.VMEM((CHUNK, hidden), jnp.float32),
            pltpu.VMEM_SHARED((n_rows, hidden), jnp.float32),
            pltpu.SemaphoreType.DMA,
            pltpu.SemaphoreType.DMA,
        ],
    )
    def gather_kernel(table_hbm, idx_hbm, out_hbm, idx_v, rows0, rows1,
                      table_sp, sem0, sem1):
        cid = lax.axis_index("c")
        sid = lax.axis_index("s")
        wid = sid * nc + cid
        base = wid * per_w

        idx_cp = pltpu.make_async_copy(idx_hbm.at[pl.ds(base, per_w)], idx_v,
                                       sem0)
        idx_cp.start()
        stripe = sid * rows_per_tile
        pltpu.sync_copy(table_hbm.at[pl.ds(stripe, rows_per_tile)],
                        table_sp.at[pl.ds(stripe, rows_per_tile)])
        plsc.subcore_barrier()
        idx_cp.wait()

        bufs = (rows0, rows1)
        sems = (sem0, sem1)

        def start_gather(i, buf, sem):
            pltpu.async_copy(table_sp.at[idx_v.at[pl.ds(i * CHUNK, CHUNK)]],
                             buf, sem)

        def wait_gather(i, buf, sem):
            pltpu.make_async_copy(
                table_sp.at[idx_v.at[pl.ds(i * CHUNK, CHUNK)]], buf, sem
            ).wait()

        def start_out(i, buf, sem):
            pltpu.async_copy(buf, out_hbm.at[pl.ds(base + i * CHUNK, CHUNK)],
                             sem)

        def wait_out(i, buf, sem):
            pltpu.make_async_copy(
                buf, out_hbm.at[pl.ds(base + i * CHUNK, CHUNK)], sem
            ).wait()

        start_gather(0, bufs[0], sems[0])

        def pair_body(p, carry):
            for b in range(2):
                i = 2 * p + b
                buf, alt = bufs[b], bufs[1 - b]
                sem, alt_sem = sems[b], sems[1 - b]
                wait_gather(i, buf, sem)
                start_out(i, buf, sem)
                if b == 0:
                    @pl.when(p > 0)
                    def _():
                        wait_out(i - 1, alt, alt_sem)
                        start_gather(i + 1, alt, alt_sem)

                    @pl.when(p == 0)
                    def _():
                        start_gather(i + 1, alt, alt_sem)
                else:
                    wait_out(i - 1, alt, alt_sem)

                    @pl.when(i + 1 < n_chunks)
                    def _():
                        start_gather(i + 1, alt, alt_sem)
            return carry

        lax.fori_loop(0, n_chunks // 2, pair_body, 0)
        wait_out(n_chunks - 1, bufs[1], sems[1])

    return gather_kernel


def kernel(top_vecs, position_ids, pos_table):
    del top_vecs
    b, s = position_ids.shape
    idx = position_ids.reshape(-1).astype(jnp.int32)
    out = _build_gather(b * s, pos_table.shape[0], pos_table.shape[1])(
        pos_table, idx)
    return out.reshape(b, s, pos_table.shape[1])
